# baseline (speedup 1.0000x reference)
import functools

import jax
import jax.numpy as jnp
from jax import lax
from jax.experimental import pallas as pl
from jax.experimental.pallas import tpu as pltpu
from jax.experimental.pallas import tpu_sc as plsc

N = 100000
K = 27
C = 16

CB = 128
NCHUNK = -(-N // CB)
NW = 32
LASTV = N - (NCHUNK - 1) * CB
BASE_CNT = NCHUNK // NW
EXTRA = NCHUNK % NW
MAXC = BASE_CNT + 1

PLANE = N * 8
KA = 16
KB = K - KA
BN = 2000


def _mm_body(x_ref, w_ref, o_ref):
    o_ref[...] = jnp.dot(x_ref[...], w_ref[...],
                         preferred_element_type=jnp.float32)[None]


def _tc_matmul(x, wcat, nt):
    return pl.pallas_call(
        _mm_body,
        grid=(N // BN, nt),
        in_specs=[pl.BlockSpec((BN, C), lambda i, t: (i, 0)),
                  pl.BlockSpec((C, 8 * C), lambda i, t: (0, t))],
        out_specs=pl.BlockSpec((1, BN, 8 * C), lambda i, t: (t, i, 0)),
        out_shape=jax.ShapeDtypeStruct((nt, N, 8 * C), jnp.float32),
    )(x, wcat)


_MESH = plsc.VectorSubcoreMesh(core_axis_name="c", subcore_axis_name="s")


def _make_sc_kernel(ng, k_lo):
    rows_n = ng * CB

    @functools.partial(
        pl.kernel,
        out_type=jax.ShapeDtypeStruct((N, C), jnp.float32),
        mesh=_MESH,
        scratch_types=[
            pltpu.VMEM((CB, K), jnp.int32),
            pltpu.VMEM((CB, K), jnp.int32),
            pltpu.VMEM((ng, CB), jnp.int32),
            pltpu.VMEM((ng, CB), jnp.int32),
            pltpu.VMEM((rows_n, C), jnp.float32),
            pltpu.VMEM((rows_n, C), jnp.float32),
            pltpu.VMEM((CB, C), jnp.float32),
            pltpu.VMEM((CB, C), jnp.float32),
            pltpu.SemaphoreType.DMA,
            pltpu.SemaphoreType.DMA,
        ],
        compiler_params=pltpu.CompilerParams(use_tc_tiling_on_sc=False,
                                             needs_layout_passes=False),
    )
    def sc_kernel(ztab, nbr, acc_h, out, raw0, raw1, idx0, idx1,
                  rows0, rows1, acc_v, out_v, sem0, sem1):
        w = lax.axis_index("s") * 2 + lax.axis_index("c")
        start = w * BASE_CNT + jnp.minimum(w, EXTRA)
        cnt = BASE_CNT + jnp.where(w < EXTRA, 1, 0)
        iot = lax.iota(jnp.int32, 16)

        def prep(i, raw_v, idx_v, rows_v, sem):
            c = start + i

            @pl.when(c < NCHUNK - 1)
            def _():
                pltpu.sync_copy(nbr.at[pl.ds(c * CB, CB)], raw_v)

            @pl.when(c == NCHUNK - 1)
            def _():
                pltpu.sync_copy(nbr.at[pl.ds(c * CB, LASTV)],
                                raw_v.at[pl.ds(0, LASTV)])

            def krow(gi, carry):
                col = k_lo + gi
                off = lax.shift_right_logical(gi, 3) * PLANE \
                    + lax.bitwise_and(gi, 7)
                for t in range(CB // 16):
                    v = plsc.load_gather(raw_v, [t * 16 + iot, iot * 0 + col])
                    idx_v[gi, pl.ds(t * 16, 16)] = \
                        lax.shift_left(v, 3) + off
                return carry

            lax.fori_loop(0, ng, krow, 0)

            def fire(gi, carry):
                pltpu.async_copy(ztab.at[idx_v.at[gi]],
                                 rows_v.at[pl.ds(gi * CB, CB)], sem)
                return carry

            lax.fori_loop(0, ng, fire, 0)

        def consume(i, rows_v, sem):
            c = start + i

            @pl.when(c < NCHUNK - 1)
            def _():
                pltpu.sync_copy(acc_h.at[pl.ds(c * CB, CB)], acc_v)

            @pl.when(c == NCHUNK - 1)
            def _():
                pltpu.sync_copy(acc_h.at[pl.ds(c * CB, LASTV)],
                                acc_v.at[pl.ds(0, LASTV)])

            pltpu.make_async_copy(ztab.at[pl.ds(0, rows_n)], rows_v,
                                  sem).wait()

            def node(j, carry):
                acc = acc_v[j, :] + rows_v[j, :]
                for gi in range(1, ng):
                    acc = acc + rows_v[gi * CB + j, :]
                out_v[j, :] = acc
                return carry

            lax.fori_loop(0, CB, node, 0)

            @pl.when(c < NCHUNK - 1)
            def _():
                pltpu.sync_copy(out_v, out.at[pl.ds(c * CB, CB)])

            @pl.when(c == NCHUNK - 1)
            def _():
                pltpu.sync_copy(out_v.at[pl.ds(0, LASTV)],
                                out.at[pl.ds(c * CB, LASTV)])

        prep(0, raw0, idx0, rows0, sem0)

        def pipe(t, carry):
            i0 = t * 2
            i1 = i0 + 1

            @pl.when(i1 < cnt)
            def _():
                prep(i1, raw1, idx1, rows1, sem1)

            @pl.when(i0 < cnt)
            def _():
                consume(i0, rows0, sem0)

            @pl.when(i1 + 1 < cnt)
            def _():
                prep(i1 + 1, raw0, idx0, rows0, sem0)

            @pl.when(i1 < cnt)
            def _():
                consume(i1, rows1, sem1)

            return carry

        lax.fori_loop(0, (MAXC + 1) // 2, pipe, 0)

    return sc_kernel


_SC_A = _make_sc_kernel(KA, 0)
_SC_B = _make_sc_kernel(KB, KA)


def kernel(input, neighbor_idx, weight, bias):
    wpad = jnp.pad(weight, ((0, 32 - K), (0, 0), (0, 0)))
    wcat = jnp.transpose(wpad, (1, 0, 2)).reshape(C, 32 * C)
    za = _tc_matmul(input, wcat[:, :KA * C], 2)
    zb = _tc_matmul(input, wcat[:, KA * C:], 2)
    ztab_a = za.reshape(2 * PLANE, C)
    ztab_b = zb.reshape(2 * PLANE, C)
    bias_rep = jnp.broadcast_to(bias, (N, C))
    out_a = _SC_A(ztab_a, neighbor_idx, bias_rep)
    return _SC_B(ztab_b, neighbor_idx, out_a)


# speedup vs baseline: 1.0283x; 1.0283x over previous
import functools

import jax
import jax.numpy as jnp
from jax import lax
from jax.experimental import pallas as pl
from jax.experimental.pallas import tpu as pltpu
from jax.experimental.pallas import tpu_sc as plsc

N = 100000
K = 27
C = 16

CB = 128
NCHUNK = -(-N // CB)
NW = 32
LASTV = N - (NCHUNK - 1) * CB
BASE_CNT = NCHUNK // NW
EXTRA = NCHUNK % NW
MAXC = BASE_CNT + 1

PLANE = N * 8
KA = 16
KB = K - KA
BN = 2000


def _mm_body(x_ref, w_ref, o_ref):
    o_ref[...] = jnp.dot(x_ref[...], w_ref[...],
                         preferred_element_type=jnp.float32)[None]


def _tc_matmul(x, wcat, nt):
    return pl.pallas_call(
        _mm_body,
        grid=(N // BN, nt),
        in_specs=[pl.BlockSpec((BN, C), lambda i, t: (i, 0)),
                  pl.BlockSpec((C, 8 * C), lambda i, t: (0, t))],
        out_specs=pl.BlockSpec((1, BN, 8 * C), lambda i, t: (t, i, 0)),
        out_shape=jax.ShapeDtypeStruct((nt, N, 8 * C), jnp.float32),
    )(x, wcat)


_MESH = plsc.VectorSubcoreMesh(core_axis_name="c", subcore_axis_name="s")


def _make_sc_kernel(ng, k_lo):
    rows_n = ng * CB

    @functools.partial(
        pl.kernel,
        out_type=jax.ShapeDtypeStruct((N, C), jnp.float32),
        mesh=_MESH,
        scratch_types=[
            pltpu.VMEM((CB, K), jnp.int32),
            pltpu.VMEM((CB, K), jnp.int32),
            pltpu.VMEM((ng, CB), jnp.int32),
            pltpu.VMEM((ng, CB), jnp.int32),
            pltpu.VMEM((rows_n, C), jnp.float32),
            pltpu.VMEM((rows_n, C), jnp.float32),
            pltpu.VMEM((C,), jnp.float32),
            pltpu.VMEM((CB, C), jnp.float32),
            pltpu.SemaphoreType.DMA,
            pltpu.SemaphoreType.DMA,
        ],
        compiler_params=pltpu.CompilerParams(use_tc_tiling_on_sc=False,
                                             needs_layout_passes=False),
    )
    def sc_kernel(ztab, nbr, bias_h, out, raw0, raw1, idx0, idx1,
                  rows0, rows1, bias_v, out_v, sem0, sem1):
        w = lax.axis_index("s") * 2 + lax.axis_index("c")
        start = w * BASE_CNT + jnp.minimum(w, EXTRA)
        cnt = BASE_CNT + jnp.where(w < EXTRA, 1, 0)
        pltpu.sync_copy(bias_h, bias_v)
        iot = lax.iota(jnp.int32, 16)

        def prep(i, raw_v, idx_v, rows_v, sem):
            c = start + i

            @pl.when(c < NCHUNK - 1)
            def _():
                pltpu.sync_copy(nbr.at[pl.ds(c * CB, CB)], raw_v)

            @pl.when(c == NCHUNK - 1)
            def _():
                pltpu.sync_copy(nbr.at[pl.ds(c * CB, LASTV)],
                                raw_v.at[pl.ds(0, LASTV)])

            def krow(gi, carry):
                col = k_lo + gi
                off = lax.shift_right_logical(gi, 3) * PLANE \
                    + lax.bitwise_and(gi, 7)
                for t in range(CB // 16):
                    v = plsc.load_gather(raw_v, [t * 16 + iot, iot * 0 + col])
                    idx_v[gi, pl.ds(t * 16, 16)] = \
                        lax.shift_left(v, 3) + off
                return carry

            lax.fori_loop(0, ng, krow, 0)

            def fire(gi, carry):
                pltpu.async_copy(ztab.at[idx_v.at[gi]],
                                 rows_v.at[pl.ds(gi * CB, CB)], sem)
                return carry

            lax.fori_loop(0, ng, fire, 0)

        def consume(i, rows_v, sem):
            c = start + i
            pltpu.make_async_copy(ztab.at[pl.ds(0, rows_n)], rows_v,
                                  sem).wait()

            def node(j, carry):
                acc = bias_v[...] + rows_v[j, :]
                for gi in range(1, ng):
                    acc = acc + rows_v[gi * CB + j, :]
                out_v[j, :] = acc
                return carry

            lax.fori_loop(0, CB, node, 0)

            @pl.when(c < NCHUNK - 1)
            def _():
                pltpu.sync_copy(out_v, out.at[pl.ds(c * CB, CB)])

            @pl.when(c == NCHUNK - 1)
            def _():
                pltpu.sync_copy(out_v.at[pl.ds(0, LASTV)],
                                out.at[pl.ds(c * CB, LASTV)])

        prep(0, raw0, idx0, rows0, sem0)

        def pipe(t, carry):
            i0 = t * 2
            i1 = i0 + 1

            @pl.when(i1 < cnt)
            def _():
                prep(i1, raw1, idx1, rows1, sem1)

            @pl.when(i0 < cnt)
            def _():
                consume(i0, rows0, sem0)

            @pl.when(i1 + 1 < cnt)
            def _():
                prep(i1 + 1, raw0, idx0, rows0, sem0)

            @pl.when(i1 < cnt)
            def _():
                consume(i1, rows1, sem1)

            return carry

        lax.fori_loop(0, (MAXC + 1) // 2, pipe, 0)

    return sc_kernel


_SC_ALL = _make_sc_kernel(K, 0)


def kernel(input, neighbor_idx, weight, bias):
    wpad = jnp.pad(weight, ((0, 32 - K), (0, 0), (0, 0)))
    wcat = jnp.transpose(wpad, (1, 0, 2)).reshape(C, 32 * C)
    z = _tc_matmul(input, wcat, 4)
    ztab = z.reshape(4 * PLANE, C)
    return _SC_ALL(ztab, neighbor_idx, bias)


# speedup vs baseline: 1.3239x; 1.2875x over previous
import functools

import jax
import jax.numpy as jnp
from jax import lax
from jax.experimental import pallas as pl
from jax.experimental.pallas import tpu as pltpu
from jax.experimental.pallas import tpu_sc as plsc

N = 100000
K = 27
C = 16

CB = 128
NPAD = 100096
NCHUNK = NPAD // CB
NW = 32
ROWS = K * CB
LASTV = N - (NCHUNK - 1) * CB
BASE_CNT = NCHUNK // NW
EXTRA = NCHUNK % NW
MAXC = BASE_CNT + 1

K2 = 32
NT = 4
PLANE = N * 8
BN = 2000


def _mm_body(x_ref, w_ref, o_ref):
    for t in range(NT):
        o_ref[t] = jnp.dot(x_ref[...], w_ref[..., t * 128:(t + 1) * 128],
                           preferred_element_type=jnp.float32)


def _tc_matmul(x, wcat):
    return pl.pallas_call(
        _mm_body,
        grid=(N // BN,),
        in_specs=[pl.BlockSpec((BN, C), lambda i: (i, 0)),
                  pl.BlockSpec((C, K2 * C), lambda i: (0, 0))],
        out_specs=pl.BlockSpec((NT, BN, 8 * C), lambda i: (0, i, 0)),
        out_shape=jax.ShapeDtypeStruct((NT, N, 8 * C), jnp.float32),
    )(x, wcat)


_MESH = plsc.VectorSubcoreMesh(core_axis_name="c", subcore_axis_name="s")


@functools.partial(
    pl.kernel,
    out_type=jax.ShapeDtypeStruct((N // 8, 8 * C), jnp.float32),
    mesh=_MESH,
    scratch_types=[
        pltpu.VMEM((K, CB), jnp.int32),
        pltpu.VMEM((K, CB), jnp.int32),
        pltpu.VMEM((ROWS, C), jnp.float32),
        pltpu.VMEM((ROWS, C), jnp.float32),
        pltpu.VMEM((CB // 8, 8 * C), jnp.float32),
        pltpu.VMEM((C,), jnp.float32),
        pltpu.SemaphoreType.DMA,
        pltpu.SemaphoreType.DMA,
    ],
    compiler_params=pltpu.CompilerParams(use_tc_tiling_on_sc=False),
)
def _sc_gather_reduce(ztab, idxn, bias_h, out, idx0, idx1, rows0, rows1,
                      out_v, bias_v, sem0, sem1):
    w = lax.axis_index("s") * 2 + lax.axis_index("c")
    start = w * BASE_CNT + jnp.minimum(w, EXTRA)
    cnt = BASE_CNT + jnp.where(w < EXTRA, 1, 0)
    pltpu.sync_copy(bias_h, bias_v)
    iot = lax.iota(jnp.int32, 16)

    def prep(i, idx_v, rows_v, sem):
        c = start + i
        pltpu.sync_copy(idxn.at[c], idx_v)

        def krow(g, carry):
            for t in range(CB // 16):
                sl = pl.ds(t * 16, 16)
                p = g * CB + t * 16 + iot
                k = lax.rem(p, K)
                idx_v[g, sl] = (lax.shift_left(idx_v[g, sl], 3)
                                + lax.shift_right_logical(k, 3) * PLANE
                                + lax.bitwise_and(k, 7))
            return carry

        lax.fori_loop(0, K, krow, 0)

        def fire(g, carry):
            pltpu.async_copy(ztab.at[idx_v.at[g]],
                             rows_v.at[pl.ds(g * CB, CB)], sem)
            return carry

        lax.fori_loop(0, K, fire, 0)

    def consume(i, rows_v, sem):
        c = start + i
        pltpu.make_async_copy(ztab.at[pl.ds(0, ROWS)], rows_v, sem).wait()

        def node(j, carry):
            base = j * K
            acc = bias_v[...] + rows_v[base, :]
            for k in range(1, K):
                acc = acc + rows_v[base + k, :]
            out_v[lax.shift_right_logical(j, 3),
                  pl.ds(lax.bitwise_and(j, 7) * C, C)] = acc
            return carry

        lax.fori_loop(0, CB, node, 0)

        @pl.when(c < NCHUNK - 1)
        def _():
            pltpu.sync_copy(out_v, out.at[pl.ds(c * (CB // 8), CB // 8)])

        @pl.when(c == NCHUNK - 1)
        def _():
            pltpu.sync_copy(out_v.at[pl.ds(0, LASTV // 8)],
                            out.at[pl.ds(c * (CB // 8), LASTV // 8)])

    prep(0, idx0, rows0, sem0)

    def pipe(t, carry):
        i0 = t * 2
        i1 = i0 + 1

        @pl.when(i1 < cnt)
        def _():
            prep(i1, idx1, rows1, sem1)

        @pl.when(i0 < cnt)
        def _():
            consume(i0, rows0, sem0)

        @pl.when(i1 + 1 < cnt)
        def _():
            prep(i1 + 1, idx0, rows0, sem0)

        @pl.when(i1 < cnt)
        def _():
            consume(i1, rows1, sem1)

        return carry

    lax.fori_loop(0, (MAXC + 1) // 2, pipe, 0)


def kernel(input, neighbor_idx, weight, bias):
    nid = jnp.pad(neighbor_idx, ((0, NPAD - N), (0, 0)))
    idxn = nid.reshape(NCHUNK, K, CB)
    wpad = jnp.pad(weight, ((0, K2 - K), (0, 0), (0, 0)))
    wcat = jnp.transpose(wpad, (1, 0, 2)).reshape(C, K2 * C)
    z3 = _tc_matmul(input, wcat)
    ztab = z3.reshape(N * K2, C)
    out8 = _sc_gather_reduce(ztab, idxn, bias)
    return out8.reshape(N, C)
